# baseline (speedup 1.0000x reference)
import functools

import jax
import jax.numpy as jnp
from jax.experimental import pallas as pl
from jax.experimental.pallas import tpu as pltpu

_N = 65536
_D = 64
_K = 1024
_E = 80
_BN = 2048
_NB = _N // _BN
_NITERS = 10


def _iter_body(x_ref, xe_ref, acc_in_ref,
               clusters_ref, acc_out_ref, cent_ref, csq_ref):
    i = pl.program_id(0)

    @pl.when(i == 0)
    def _init():
        acc = acc_in_ref[...]
        cnt = acc[:, 64:65]
        cent = acc[:, 0:64] / cnt
        cent_ref[...] = cent
        csq_ref[...] = jnp.sum(cent * cent, axis=1)[None, :]

    xs = x_ref[...]
    cent = cent_ref[...]
    x_sq = jnp.sum(xs * xs, axis=1, keepdims=True)
    xc = jax.lax.dot_general(xs, cent, (((1,), (1,)), ((), ())),
                             preferred_element_type=jnp.float32)
    scores = (x_sq - 2.0 * xc) + csq_ref[...]
    idx = jnp.argmin(scores, axis=1).astype(jnp.int32)
    clusters_ref[...] = idx
    ohT = (jax.lax.broadcasted_iota(jnp.int32, (_K, _BN), 0)
           == idx[None, :]).astype(jnp.float32)
    pacc = jax.lax.dot_general(ohT, xe_ref[...], (((1,), (0,)), ((), ())),
                               preferred_element_type=jnp.float32,
                               precision=jax.lax.Precision.HIGHEST)

    @pl.when(i == 0)
    def _first():
        acc_out_ref[...] = pacc

    @pl.when(i > 0)
    def _rest():
        acc_out_ref[...] += pacc


_iter_call = pl.pallas_call(
    _iter_body,
    grid=(_NB,),
    in_specs=[
        pl.BlockSpec((_BN, _D), lambda i: (i, 0)),
        pl.BlockSpec((_BN, _E), lambda i: (i, 0)),
        pl.BlockSpec((_K, _E), lambda i: (0, 0)),
    ],
    out_specs=[
        pl.BlockSpec((_BN,), lambda i: (i,)),
        pl.BlockSpec((_K, _E), lambda i: (0, 0)),
    ],
    out_shape=[
        jax.ShapeDtypeStruct((_N,), jnp.int32),
        jax.ShapeDtypeStruct((_K, _E), jnp.float32),
    ],
    scratch_shapes=[
        pltpu.VMEM((_K, 64), jnp.float32),
        pltpu.VMEM((1, _K), jnp.float32),
    ],
    compiler_params=pltpu.CompilerParams(
        dimension_semantics=("arbitrary",),
    ),
)


def _final_body(acc_ref, cent_ref, npts_ref):
    acc = acc_ref[...]
    cnt = acc[:, 64:65]
    cent_ref[...] = acc[:, 0:64] / cnt
    npts_ref[...] = cnt


_final_call = pl.pallas_call(
    _final_body,
    out_shape=[
        jax.ShapeDtypeStruct((_K, _D), jnp.float32),
        jax.ShapeDtypeStruct((_K, 1), jnp.float32),
    ],
)


def kernel(x):
    ones = jnp.ones((_N, 1), jnp.float32)
    zeros = jnp.zeros((_N, _E - _D - 1), jnp.float32)
    x_ext = jnp.concatenate([x, ones, zeros], axis=1)
    acc = jnp.concatenate(
        [x[:_K], jnp.ones((_K, 1), jnp.float32),
         jnp.zeros((_K, _E - _D - 1), jnp.float32)], axis=1)
    clusters = None
    for _ in range(_NITERS):
        clusters, acc = _iter_call(x, x_ext, acc)
    cent, npts = _final_call(acc)
    return clusters, cent, npts.reshape(_K)
